# Initial kernel scaffold; baseline (speedup 1.0000x reference)
import jax
import jax.numpy as jnp
from jax.experimental import pallas as pl


def kernel(input, level_w, feat_w, ch_w):
    raise NotImplementedError("write your pallas kernel here")


# speedup vs baseline: 1.4057x; 1.4057x over previous
import functools

import jax
import jax.numpy as jnp
from jax import lax
from jax.experimental import pallas as pl
from jax.experimental.pallas import tpu as pltpu
from jax.experimental.pallas import tpu_sc as plsc

MAX_VAL = 52000.0
MIN_VAL = -53000.0
NUM_LEVELS = 1000
CH = 32
T = 512
D = 2048

NUM_CORES = 2
NUM_SUBCORES = 16
NW = NUM_CORES * NUM_SUBCORES
TB = T // NW
LANES = 16
CBURST = 8


def _level_indices(xr):
    clipped = jnp.minimum(jnp.maximum(xr, MIN_VAL), MAX_VAL)
    v = (NUM_LEVELS - 1) * (clipped - MIN_VAL) / (MAX_VAL - MIN_VAL)
    r0 = v.astype(jnp.int32)
    frac = v - r0.astype(jnp.float32)
    odd = jnp.bitwise_and(r0, 1)
    up = (frac > 0.5) | ((frac == 0.5) & (odd == 1))
    idx = r0 + jnp.where(up, 1, 0)
    return jnp.minimum(jnp.maximum(idx, 0), NUM_LEVELS - 1)


@functools.partial(
    pl.kernel,
    mesh=plsc.VectorSubcoreMesh(core_axis_name="c", subcore_axis_name="s"),
    out_type=jax.ShapeDtypeStruct((NW, CH, D), jnp.float32),
    scratch_types=[
        pltpu.VMEM((CH, TB), jnp.float32),
        pltpu.VMEM((CH, TB), jnp.int32),
        pltpu.VMEM((TB, D), jnp.float32),
        pltpu.VMEM((2, TB, D), jnp.float32),
        pltpu.VMEM((CBURST, D), jnp.float32),
        pltpu.SemaphoreType.DMA,
        pltpu.SemaphoreType.DMA,
        pltpu.SemaphoreType.DMA,
        pltpu.SemaphoreType.DMA,
    ],
)
def _sc_encode(xf_hbm, level_hbm, feat_hbm, part_hbm,
               x_v, idx_v, feat_v, gbuf, obuf, gsem0, gsem1, osem, xsem):
    wid = lax.axis_index("s") * NUM_CORES + lax.axis_index("c")
    t0 = wid * TB
    gsems = (gsem0, gsem1)

    def xcopy(c):
        return pltpu.make_async_copy(
            xf_hbm.at[pl.ds(c * T + t0, TB)], x_v.at[c], xsem)

    for c in range(CH):
        xcopy(c).start()
    pltpu.sync_copy(feat_hbm.at[pl.ds(t0, TB), :], feat_v)
    for c in range(CH):
        xcopy(c).wait()

    for c in range(CH):
        idx_v[c, :] = _level_indices(x_v[c, :])

    def gcopy(c, b):
        return pltpu.make_async_copy(
            level_hbm.at[idx_v.at[c]], gbuf.at[b], gsems[b])

    def ocopy(c0):
        return pltpu.make_async_copy(
            obuf, part_hbm.at[wid, pl.ds(c0, CBURST)], osem)

    gcopy(0, 0).start()
    for c in range(CH):
        b = c & 1
        if c + 1 < CH:
            gcopy(c + 1, 1 - b).start()
        if c >= CBURST and c % CBURST == 0:
            ocopy(c - CBURST).wait()
        gcopy(c, b).wait()

        def dbody(i, _, b=b, co=c % CBURST):
            sl = pl.ds(i * LANES, LANES)
            acc = gbuf[b, 0, sl] * feat_v[0, sl]
            for tt in range(1, TB):
                acc = acc + gbuf[b, tt, sl] * feat_v[tt, sl]
            obuf[co, sl] = acc
            return 0

        lax.fori_loop(0, D // LANES, dbody, 0)
        if c % CBURST == CBURST - 1:
            ocopy(c - (CBURST - 1)).start()
    ocopy(CH - CBURST).wait()


def _tc_finish_body(part_ref, ch_ref, out_ref):
    s = jnp.sum(part_ref[...], axis=0)
    s = jnp.where(s > 0, 1.0, -1.0).astype(jnp.float32)
    bnd = s * ch_ref[...]

    def rolled(v, k):
        return jnp.concatenate([v[:, D - k:], v[:, :D - k]], axis=1)

    ng = (rolled(bnd[0:CH - 3], 3) * rolled(bnd[1:CH - 2], 2)
          * rolled(bnd[2:CH - 1], 1) * bnd[3:CH])
    o = jnp.sum(ng, axis=0, keepdims=True)
    out_ref[...] = jnp.where(o > 0, 1.0, -1.0).astype(jnp.float32)


def kernel(input, level_w, feat_w, ch_w):
    partials = _sc_encode(input.reshape(-1), level_w, feat_w)
    return pl.pallas_call(
        _tc_finish_body,
        out_shape=jax.ShapeDtypeStruct((1, D), jnp.float32),
    )(partials, ch_w)
